# baseline (speedup 1.0000x reference)
import functools

import jax
import jax.numpy as jnp
from jax import lax
from jax.experimental import pallas as pl
from jax.experimental.pallas import tpu as pltpu
from jax.experimental.pallas import tpu_sc as plsc

N_TOKENS = 100000
EMBED = 32
NNZ = 5000000
BATCH = 16384
X_MAX_INV = 0.01
ALPHA = 0.75

NC = 2
NS = 16
NW = NC * NS
B_PER_W = BATCH // NW
CHUNK = 128
NCH = B_PER_W // CHUNK
GPC = CHUNK // 16

LN2 = 0.6931471805599453
C0 = 2.8853900817779268
C1 = 0.9617966939259756
C2 = 0.5770780163555854
C3 = 0.41219858311113243


def _log2(x):
    bits = lax.bitcast_convert_type(x, jnp.int32)
    e = jnp.right_shift(bits, 23) - 127
    mbits = jnp.bitwise_or(jnp.bitwise_and(bits, 0x007FFFFF), 0x3F800000)
    m = lax.bitcast_convert_type(mbits, jnp.float32)
    big = m >= 1.4142135
    m = jnp.where(big, m * 0.5, m)
    e = e + jnp.where(big, 1, 0)
    s = (m - 1.0) / (m + 1.0)
    s2 = s * s
    t = s * (C0 + s2 * (C1 + s2 * (C2 + s2 * C3)))
    return e.astype(jnp.float32) + t


def _glove_body(idx_hbm, rows_hbm, cols_hbm, vals_hbm, emb_hbm, embt_hbm,
                out_hbm,
                idx_v, rows_v, cols_v, vals_v, w_v, wt_v,
                pbuf, accbuf, redbuf, outbuf, shared,
                sem, sem0, sem1, sem2, sem3):
    cid = lax.axis_index("c")
    sid = lax.axis_index("s")
    wid = cid * NS + sid
    csems = [sem0, sem1, sem2, sem3]

    pltpu.sync_copy(idx_hbm.at[pl.ds(wid * NCH, NCH)], idx_v)

    handles = []
    for j in range(NCH):
        handles.append(pltpu.async_copy(rows_hbm.at[idx_v.at[j]], rows_v.at[j], sem))
        handles.append(pltpu.async_copy(cols_hbm.at[idx_v.at[j]], cols_v.at[j], sem))
        handles.append(pltpu.async_copy(vals_hbm.at[idx_v.at[j]],
                                        vals_v.at[pl.ds(j * CHUNK, CHUNK)], sem))
    for h in handles:
        h.wait()

    chunk_handles = []
    for j in range(NCH):
        sl = pl.ds(j * CHUNK, CHUNK)
        cs = csems[j]
        chunk_handles.append((
            pltpu.async_copy(emb_hbm.at[rows_v.at[j]], w_v.at[sl], cs),
            pltpu.async_copy(embt_hbm.at[cols_v.at[j]], wt_v.at[sl], cs),
        ))

    lane = lax.iota(jnp.int32, 16)
    lane16 = lane * 16
    tailmask = jnp.where(lane >= 12, 1.0, 0.0).astype(jnp.float32)

    def group(g, acc):
        b0 = g * 16
        for l in range(16):
            b = b0 + l
            p = (w_v[b, pl.ds(0, 16)] * wt_v[b, pl.ds(0, 16)]
                 + w_v[b, pl.ds(16, 16)] * wt_v[b, pl.ds(16, 16)]
                 + tailmask * (w_v[b, pl.ds(20, 16)] * wt_v[b, pl.ds(20, 16)]))
            pbuf[pl.ds(l * 16, 16)] = p
        dots = plsc.load_gather(pbuf, [lane16])
        for c in range(1, 16):
            dots = dots + plsc.load_gather(pbuf, [lane16 + c])
        cval = vals_v[pl.ds(b0, 16)]
        logc = _log2(cval) * LN2
        y = jnp.minimum(cval * X_MAX_INV, 1.0)
        wgt = jnp.exp((ALPHA * LN2) * _log2(y))
        r = dots - logc
        return acc + wgt * r * r

    acc = jnp.zeros((16,), jnp.float32)
    for j in range(NCH):
        for h in chunk_handles[j]:
            h.wait()
        acc = lax.fori_loop(j * GPC, (j + 1) * GPC, group, acc)

    accbuf[...] = acc * 0.5
    pltpu.sync_copy(accbuf, shared.at[sid])
    plsc.subcore_barrier()

    @pl.when(sid == 0)
    def _():
        pltpu.sync_copy(shared, redbuf)
        tot = redbuf[0, :]
        for srow in range(1, NS):
            tot = tot + redbuf[srow, :]
        total = jnp.sum(tot)
        outbuf[...] = jnp.full((16,), total)
        pltpu.sync_copy(outbuf, out_hbm.at[cid])


@jax.jit
def kernel(indices, coo_rows, coo_cols, coo_vals, embedding, embedding_tilde,
           bias, bias_tilde):
    idx2d = indices.reshape(BATCH // CHUNK, CHUNK).astype(jnp.int32)
    ones = jnp.ones((N_TOKENS, 1), jnp.float32)
    zeros2 = jnp.zeros((N_TOKENS, 2), jnp.float32)
    emb_aug = jnp.concatenate([embedding, bias, ones, zeros2], axis=1)
    embt_aug = jnp.concatenate([embedding_tilde, ones, bias_tilde, zeros2], axis=1)

    mesh = plsc.VectorSubcoreMesh(core_axis_name="c", subcore_axis_name="s",
                                  num_cores=NC, num_subcores=NS)
    run = pl.kernel(
        _glove_body,
        out_type=jax.ShapeDtypeStruct((NC, 16), jnp.float32),
        mesh=mesh,
        compiler_params=pltpu.CompilerParams(needs_layout_passes=False,
                                             use_tc_tiling_on_sc=False),
        scratch_types=[
            pltpu.VMEM((NCH, CHUNK), jnp.int32),
            pltpu.VMEM((NCH, CHUNK), jnp.int32),
            pltpu.VMEM((NCH, CHUNK), jnp.int32),
            pltpu.VMEM((B_PER_W,), jnp.float32),
            pltpu.VMEM((B_PER_W, EMBED + 4), jnp.float32),
            pltpu.VMEM((B_PER_W, EMBED + 4), jnp.float32),
            pltpu.VMEM((256,), jnp.float32),
            pltpu.VMEM((16,), jnp.float32),
            pltpu.VMEM((NS, 16), jnp.float32),
            pltpu.VMEM((16,), jnp.float32),
            pltpu.VMEM_SHARED((NS, 16), jnp.float32),
            pltpu.SemaphoreType.DMA,
            pltpu.SemaphoreType.DMA,
            pltpu.SemaphoreType.DMA,
            pltpu.SemaphoreType.DMA,
            pltpu.SemaphoreType.DMA,
        ],
    )
    partials = run(idx2d, coo_rows, coo_cols, coo_vals, emb_aug, embt_aug)
    return partials[0, 0] + partials[1, 0]


# speedup vs baseline: 1.6773x; 1.6773x over previous
import functools

import jax
import jax.numpy as jnp
from jax import lax
from jax.experimental import pallas as pl
from jax.experimental.pallas import tpu as pltpu
from jax.experimental.pallas import tpu_sc as plsc

N_TOKENS = 100000
EMBED = 32
NNZ = 5000000
BATCH = 16384
X_MAX_INV = 0.01
ALPHA = 0.75

NC = 2
NS = 16
NW = NC * NS
B_PER_W = BATCH // NW
CHUNK = 128
NCH = B_PER_W // CHUNK
GPC = CHUNK // 16

LN2 = 0.6931471805599453
C0 = 2.8853900817779268
C1 = 0.9617966939259756
C2 = 0.5770780163555854
C3 = 0.41219858311113243


def _log2(x):
    bits = lax.bitcast_convert_type(x, jnp.int32)
    e = jnp.right_shift(bits, 23) - 127
    mbits = jnp.bitwise_or(jnp.bitwise_and(bits, 0x007FFFFF), 0x3F800000)
    m = lax.bitcast_convert_type(mbits, jnp.float32)
    big = m >= 1.4142135
    m = jnp.where(big, m * 0.5, m)
    e = e + jnp.where(big, 1, 0)
    s = (m - 1.0) / (m + 1.0)
    s2 = s * s
    t = s * (C0 + s2 * (C1 + s2 * (C2 + s2 * C3)))
    return e.astype(jnp.float32) + t


def _glove_body(idx_hbm, rows_hbm, cols_hbm, vals_hbm, emb_hbm, embt_hbm,
                bias_hbm, biast_hbm, out_hbm,
                idx_v, rows_v, cols_v, vals_v, w_v, wt_v, b1_v, bt1_v,
                pbuf, accbuf, redbuf, outbuf, shared,
                sem, sem0, sem1, sem2, sem3):
    cid = lax.axis_index("c")
    sid = lax.axis_index("s")
    wid = cid * NS + sid
    csems = [sem0, sem1, sem2, sem3]

    pltpu.sync_copy(idx_hbm.at[pl.ds(wid * NCH, NCH)], idx_v)

    handles = []
    for j in range(NCH):
        handles.append(pltpu.async_copy(rows_hbm.at[idx_v.at[j]], rows_v.at[j], sem))
        handles.append(pltpu.async_copy(cols_hbm.at[idx_v.at[j]], cols_v.at[j], sem))
        handles.append(pltpu.async_copy(vals_hbm.at[idx_v.at[j]],
                                        vals_v.at[pl.ds(j * CHUNK, CHUNK)], sem))
    for h in handles:
        h.wait()

    chunk_handles = []
    for j in range(NCH):
        sl = pl.ds(j * CHUNK, CHUNK)
        cs = csems[j]
        chunk_handles.append((
            pltpu.async_copy(emb_hbm.at[rows_v.at[j]], w_v.at[sl], cs),
            pltpu.async_copy(embt_hbm.at[cols_v.at[j]], wt_v.at[sl], cs),
            pltpu.async_copy(bias_hbm.at[rows_v.at[j]], b1_v.at[sl], cs),
            pltpu.async_copy(biast_hbm.at[cols_v.at[j]], bt1_v.at[sl], cs),
        ))

    lane = lax.iota(jnp.int32, 16)
    lane16 = lane * 16

    def group(g, acc):
        b0 = g * 16
        for l in range(16):
            b = b0 + l
            we, wo = plsc.unpack(w_v[b, :], format=plsc.PackFormat.INTERLEAVED)
            te, to = plsc.unpack(wt_v[b, :], format=plsc.PackFormat.INTERLEAVED)
            p = we * te + wo * to
            pbuf[pl.ds(l * 16, 16)] = p
        dots = plsc.load_gather(pbuf, [lane16])
        for c in range(1, 16):
            dots = dots + plsc.load_gather(pbuf, [lane16 + c])
        dots = dots + b1_v[pl.ds(b0, 16)] + bt1_v[pl.ds(b0, 16)]
        cval = vals_v[pl.ds(b0, 16)]
        logc = _log2(cval) * LN2
        y = jnp.minimum(cval * X_MAX_INV, 1.0)
        wgt = jnp.exp((ALPHA * LN2) * _log2(y))
        r = dots - logc
        return acc + wgt * r * r

    acc = jnp.zeros((16,), jnp.float32)
    for j in range(NCH):
        for h in chunk_handles[j]:
            h.wait()
        acc = lax.fori_loop(j * GPC, (j + 1) * GPC, group, acc)

    accbuf[...] = acc * 0.5
    pltpu.sync_copy(accbuf, shared.at[sid])
    plsc.subcore_barrier()

    @pl.when(sid == 0)
    def _():
        pltpu.sync_copy(shared, redbuf)
        tot = redbuf[0, :]
        for srow in range(1, NS):
            tot = tot + redbuf[srow, :]
        total = jnp.sum(tot)
        outbuf[...] = jnp.full((16,), total)
        pltpu.sync_copy(outbuf, out_hbm.at[cid])


@jax.jit
def kernel(indices, coo_rows, coo_cols, coo_vals, embedding, embedding_tilde,
           bias, bias_tilde):
    idx2d = indices.reshape(BATCH // CHUNK, CHUNK).astype(jnp.int32)

    mesh = plsc.VectorSubcoreMesh(core_axis_name="c", subcore_axis_name="s",
                                  num_cores=NC, num_subcores=NS)
    run = pl.kernel(
        _glove_body,
        out_type=jax.ShapeDtypeStruct((NC, 16), jnp.float32),
        mesh=mesh,
        compiler_params=pltpu.CompilerParams(needs_layout_passes=False,
                                             use_tc_tiling_on_sc=False),
        scratch_types=[
            pltpu.VMEM((NCH, CHUNK), jnp.int32),
            pltpu.VMEM((NCH, CHUNK), jnp.int32),
            pltpu.VMEM((NCH, CHUNK), jnp.int32),
            pltpu.VMEM((B_PER_W,), jnp.float32),
            pltpu.VMEM((B_PER_W, EMBED), jnp.bfloat16),
            pltpu.VMEM((B_PER_W, EMBED), jnp.bfloat16),
            pltpu.VMEM((B_PER_W,), jnp.float32),
            pltpu.VMEM((B_PER_W,), jnp.float32),
            pltpu.VMEM((256,), jnp.float32),
            pltpu.VMEM((16,), jnp.float32),
            pltpu.VMEM((NS, 16), jnp.float32),
            pltpu.VMEM((16,), jnp.float32),
            pltpu.VMEM_SHARED((NS, 16), jnp.float32),
            pltpu.SemaphoreType.DMA,
            pltpu.SemaphoreType.DMA,
            pltpu.SemaphoreType.DMA,
            pltpu.SemaphoreType.DMA,
            pltpu.SemaphoreType.DMA,
        ],
    )
    partials = run(idx2d, coo_rows, coo_cols, coo_vals,
                   embedding.astype(jnp.bfloat16),
                   embedding_tilde.astype(jnp.bfloat16),
                   bias.T.reshape(N_TOKENS), bias_tilde.T.reshape(N_TOKENS))
    return partials[0, 0] + partials[1, 0]


# speedup vs baseline: 2.2566x; 1.3454x over previous
import functools

import jax
import jax.numpy as jnp
from jax import lax
from jax.experimental import pallas as pl
from jax.experimental.pallas import tpu as pltpu
from jax.experimental.pallas import tpu_sc as plsc

N_TOKENS = 100000
EMBED = 32
NNZ = 5000000
BATCH = 16384
X_MAX_INV = 0.01
ALPHA = 0.75

NC = 2
NS = 16
NW = NC * NS
B_PER_W = BATCH // NW
CHUNK = 128
NCH = B_PER_W // CHUNK
GROUPS = B_PER_W // 16

LN2 = 0.6931471805599453
C0 = 2.8853900817779268
C1 = 0.9617966939259756
C2 = 0.5770780163555854
C3 = 0.41219858311113243


def _log2(x):
    bits = lax.bitcast_convert_type(x, jnp.int32)
    e = jnp.right_shift(bits, 23) - 127
    mbits = jnp.bitwise_or(jnp.bitwise_and(bits, 0x007FFFFF), 0x3F800000)
    m = lax.bitcast_convert_type(mbits, jnp.float32)
    big = m >= 1.4142135
    m = jnp.where(big, m * 0.5, m)
    e = e + jnp.where(big, 1, 0)
    s = (m - 1.0) / (m + 1.0)
    s2 = s * s
    t = s * (C0 + s2 * (C1 + s2 * (C2 + s2 * C3)))
    return e.astype(jnp.float32) + t


def _glove_body(idx_hbm, rows_hbm, cols_hbm, vals_hbm, emb_hbm, embt_hbm,
                bias_hbm, biast_hbm, out_hbm,
                idx_v, rows_v, cols_v, vals_v, w_v, wt_v, b_v, bt_v,
                pbuf, accbuf, redbuf, outbuf, shared, sem):
    cid = lax.axis_index("c")
    sid = lax.axis_index("s")
    wid = cid * NS + sid

    pltpu.sync_copy(idx_hbm.at[pl.ds(wid * NCH, NCH)], idx_v)

    handles = []
    for j in range(NCH):
        handles.append(pltpu.async_copy(rows_hbm.at[idx_v.at[j]], rows_v.at[j], sem))
        handles.append(pltpu.async_copy(cols_hbm.at[idx_v.at[j]], cols_v.at[j], sem))
        handles.append(pltpu.async_copy(vals_hbm.at[idx_v.at[j]],
                                        vals_v.at[pl.ds(j * CHUNK, CHUNK)], sem))
    for h in handles:
        h.wait()

    handles = []
    for j in range(NCH):
        sl = pl.ds(j * CHUNK, CHUNK)
        handles.append(pltpu.async_copy(emb_hbm.at[rows_v.at[j]], w_v.at[sl], sem))
        handles.append(pltpu.async_copy(embt_hbm.at[cols_v.at[j]], wt_v.at[sl], sem))
        handles.append(pltpu.async_copy(bias_hbm.at[rows_v.at[j]], b_v.at[sl], sem))
        handles.append(pltpu.async_copy(biast_hbm.at[cols_v.at[j]], bt_v.at[sl], sem))
    for h in handles:
        h.wait()

    lane = lax.iota(jnp.int32, 16)
    lane16 = lane * 16

    def group(g, acc):
        b0 = g * 16
        for l in range(16):
            b = b0 + l
            p = (w_v[b, pl.ds(0, 16)] * wt_v[b, pl.ds(0, 16)]
                 + w_v[b, pl.ds(16, 16)] * wt_v[b, pl.ds(16, 16)])
            pbuf[pl.ds(l * 16, 16)] = p
        dots = plsc.load_gather(pbuf, [lane16])
        for c in range(1, 16):
            dots = dots + plsc.load_gather(pbuf, [lane16 + c])
        dots = dots + b_v[pl.ds(b0, 16)] + bt_v[pl.ds(b0, 16)]
        cval = vals_v[pl.ds(b0, 16)]
        logc = _log2(cval) * LN2
        y = jnp.minimum(cval * X_MAX_INV, 1.0)
        wgt = jnp.exp((ALPHA * LN2) * _log2(y))
        r = dots - logc
        return acc + wgt * r * r

    acc = lax.fori_loop(0, GROUPS, group, jnp.zeros((16,), jnp.float32))

    accbuf[...] = acc * 0.5
    pltpu.sync_copy(accbuf, shared.at[sid])
    plsc.subcore_barrier()

    @pl.when(sid == 0)
    def _():
        pltpu.sync_copy(shared, redbuf)
        tot = redbuf[0, :]
        for srow in range(1, NS):
            tot = tot + redbuf[srow, :]
        total = jnp.sum(tot)
        outbuf[...] = jnp.full((16,), total)
        pltpu.sync_copy(outbuf, out_hbm.at[cid])


@jax.jit
def kernel(indices, coo_rows, coo_cols, coo_vals, embedding, embedding_tilde,
           bias, bias_tilde):
    idx2d = indices.reshape(BATCH // CHUNK, CHUNK).astype(jnp.int32)
    bias_f = bias.reshape(N_TOKENS)
    biast_f = bias_tilde.reshape(N_TOKENS)

    mesh = plsc.VectorSubcoreMesh(core_axis_name="c", subcore_axis_name="s",
                                  num_cores=NC, num_subcores=NS)
    run = pl.kernel(
        _glove_body,
        out_type=jax.ShapeDtypeStruct((NC, 16), jnp.float32),
        mesh=mesh,
        compiler_params=pltpu.CompilerParams(needs_layout_passes=False,
                                             use_tc_tiling_on_sc=False),
        scratch_types=[
            pltpu.VMEM((NCH, CHUNK), jnp.int32),
            pltpu.VMEM((NCH, CHUNK), jnp.int32),
            pltpu.VMEM((NCH, CHUNK), jnp.int32),
            pltpu.VMEM((B_PER_W,), jnp.float32),
            pltpu.VMEM((B_PER_W, EMBED), jnp.float32),
            pltpu.VMEM((B_PER_W, EMBED), jnp.float32),
            pltpu.VMEM((B_PER_W,), jnp.float32),
            pltpu.VMEM((B_PER_W,), jnp.float32),
            pltpu.VMEM((256,), jnp.float32),
            pltpu.VMEM((16,), jnp.float32),
            pltpu.VMEM((NS, 16), jnp.float32),
            pltpu.VMEM((16,), jnp.float32),
            pltpu.VMEM_SHARED((NS, 16), jnp.float32),
            pltpu.SemaphoreType.DMA,
        ],
    )
    partials = run(idx2d, coo_rows, coo_cols, coo_vals, embedding,
                   embedding_tilde, bias_f, biast_f)
    return partials[0, 0] + partials[1, 0]
